# baseline (speedup 1.0000x reference)
import functools

import jax
import jax.numpy as jnp
from jax import lax
from jax.experimental import pallas as pl
from jax.experimental.pallas import tpu as pltpu
from jax.experimental.pallas import tpu_sc as plsc

VOCAB = 1000000
MAX_SEQ = 2048
HID = 64
B = 4096
L = 200

NUM_CORES = 2
NUM_SUBCORES = 16
NUM_WORKERS = NUM_CORES * NUM_SUBCORES
BBLK = B // NUM_WORKERS
NSLOT = 2
PL = 2


def _sc_body(xt_hbm, tok_hbm, pos_hbm, out_hbm, idx_s, idx_v, pos_v, rows_v,
             btile_v, g0, g1, s0, s1):
    g_sems = (g0, g1)
    s_sems = (s0, s1)
    wid = lax.axis_index("s") * NUM_CORES + lax.axis_index("c")

    pltpu.sync_copy(pos_hbm.at[pl.ds(0, L)], pos_v)
    for half_l in range(2):
        pltpu.sync_copy(
            xt_hbm.at[pl.ds(half_l * (L // 2), L // 2),
                      pl.ds(wid * BBLK, BBLK)], idx_s)

        def repack_body(r, _):
            dst_row = (half_l * (L // 2) + r) // PL
            dst_off = lax.rem(half_l * (L // 2) + r, PL) * BBLK
            for c in range(BBLK // 16):
                sl = pl.ds(c * 16, 16)
                idx_v[dst_row, pl.ds(dst_off + c * 16, 16)] = idx_s[r, sl]
            return 0
        lax.fori_loop(0, L // 2, repack_body, 0)

    def issue_gather(step, slot):
        pltpu.async_copy(tok_hbm.at[idx_v.at[step]],
                         rows_v.at[pl.ds(slot * PL * BBLK, PL * BBLK)],
                         g_sems[slot])

    def wait_gather(slot):
        pltpu.make_async_copy(tok_hbm.at[idx_v.at[0]],
                              rows_v.at[pl.ds(slot * PL * BBLK, PL * BBLK)],
                              g_sems[slot]).wait()

    def issue_store(l, slot, half):
        pltpu.async_copy(
            btile_v.at[pl.ds((slot * PL + half) * 8, 8), :, :, :],
            out_hbm.at[l, :, pl.ds(wid, 1), :, :], s_sems[slot])

    def wait_store(slot):
        for _half in range(PL):
            pltpu.make_async_copy(btile_v.at[pl.ds(0, 8), :, :, :],
                                  out_hbm.at[0, :, pl.ds(0, 1), :, :],
                                  s_sems[slot]).wait()

    lane = lax.broadcasted_iota(jnp.int32, (16,), 0)
    zeros16 = lane * 0

    def compute(l, slot, half):
        rbase = (slot * PL + half) * BBLK
        brow = (slot * PL + half) * 8
        pos_regs = [pos_v[l, pl.ds(k * 16, 16)] for k in range(HID // 16)]

        def blk_body(t, _):
            jj = lax.shift_right_logical(t, 4)
            d = lax.bitwise_and(t, 15)
            jvec = jj * 16 + lane
            rowvec = rbase + jvec
            perm = lax.bitwise_and(lane + d, 15)
            perm_hi = lax.shift_right_logical(perm, 3)
            perm_lo = lax.bitwise_and(perm, 7)
            for k in range(HID // 16):
                pos_d = jnp.take(pos_regs[k], perm, unique_indices=True,
                                 indices_are_sorted=False, mode="fill")
                v = plsc.load_gather(rows_v, [rowvec, k * 16 + perm])
                plsc.store_scatter(
                    btile_v,
                    [brow + k * 2 + perm_hi, zeros16, perm_lo, jvec],
                    v + pos_d)
            return 0
        lax.fori_loop(0, (BBLK // 16) * 16, blk_body, 0, unroll=2)

    issue_gather(0, 0)

    NSTEP = L // PL

    def macro_body(i, _):
        for p in range(2):
            step = 2 * i + p
            slot = p
            other = 1 - p
            wait_gather(slot)
            @pl.when(i > 0)
            def _():
                wait_store(other)
            @pl.when(step + 1 < NSTEP)
            def _():
                issue_gather(step + 1, other)
            for half in range(PL):
                l = step * PL + half
                compute(l, slot, half)
                issue_store(l, slot, half)
        return 0

    lax.fori_loop(0, NSTEP // 2, macro_body, 0)

    for slot in range(NSLOT):
        wait_store(slot)


@jax.jit
def _tpe(xt, tok_table, pos_table):
    mesh = plsc.VectorSubcoreMesh(core_axis_name="c", subcore_axis_name="s")
    kern = functools.partial(
        pl.kernel,
        mesh=mesh,
        out_type=jax.ShapeDtypeStruct((L, 8, NUM_WORKERS, 8, 128),
                                      jnp.float32),
        scratch_types=[
            pltpu.VMEM((L // 2, BBLK), jnp.int32),
            pltpu.VMEM((L // PL, PL * BBLK), jnp.int32),
            pltpu.VMEM((L, HID), jnp.float32),
            pltpu.VMEM((NSLOT * PL * BBLK, HID), jnp.float32),
            pltpu.VMEM((NSLOT * PL * 8, 1, 8, 128), jnp.float32),
            pltpu.SemaphoreType.DMA,
            pltpu.SemaphoreType.DMA,
            pltpu.SemaphoreType.DMA,
            pltpu.SemaphoreType.DMA,
        ],
        compiler_params=pltpu.CompilerParams(use_tc_tiling_on_sc=False,
                                             needs_layout_passes=False),
    )(_sc_body)
    return kern(xt, tok_table, pos_table)


def kernel(x, tok_table, pos_table):
    xt = x.T.astype(jnp.int32)
    out5 = _tpe(xt, tok_table, pos_table)
    return out5.transpose(2, 4, 0, 1, 3).reshape(B, L, HID)


# speedup vs baseline: 1.0602x; 1.0602x over previous
import functools

import jax
import jax.numpy as jnp
from jax import lax
from jax.experimental import pallas as pl
from jax.experimental.pallas import tpu as pltpu
from jax.experimental.pallas import tpu_sc as plsc

VOCAB = 1000000
MAX_SEQ = 2048
HID = 64
B = 4096
L = 200

NUM_CORES = 2
NUM_SUBCORES = 16
NUM_WORKERS = NUM_CORES * NUM_SUBCORES
BBLK = B // NUM_WORKERS
NSLOT = 2
PL = 2


def _sc_body(xt_hbm, tok_hbm, pos_hbm, out_hbm, idx_s, idx_v, pos_v, rows_v,
             btile_v, g0, g1, s0, s1):
    g_sems = (g0, g1)
    s_sems = (s0, s1)
    wid = lax.axis_index("s") * NUM_CORES + lax.axis_index("c")

    pltpu.sync_copy(pos_hbm.at[pl.ds(0, L)], pos_v)
    for half_l in range(2):
        pltpu.sync_copy(
            xt_hbm.at[pl.ds(half_l * (L // 2), L // 2),
                      pl.ds(wid * BBLK, BBLK)], idx_s)

        def repack_body(r, _):
            dst_row = (half_l * (L // 2) + r) // PL
            dst_off = lax.rem(half_l * (L // 2) + r, PL) * BBLK
            for c in range(BBLK // 16):
                sl = pl.ds(c * 16, 16)
                idx_v[dst_row, pl.ds(dst_off + c * 16, 16)] = idx_s[r, sl]
            return 0
        lax.fori_loop(0, L // 2, repack_body, 0)

    def issue_gather(step, slot):
        pltpu.async_copy(tok_hbm.at[idx_v.at[step]],
                         rows_v.at[pl.ds(slot * PL * BBLK, PL * BBLK)],
                         g_sems[slot])

    def wait_gather(slot):
        pltpu.make_async_copy(tok_hbm.at[idx_v.at[0]],
                              rows_v.at[pl.ds(slot * PL * BBLK, PL * BBLK)],
                              g_sems[slot]).wait()

    def issue_store(l, slot, half):
        pltpu.async_copy(
            btile_v.at[pl.ds((slot * PL + half) * 8, 8), :, :, :],
            out_hbm.at[l, :, pl.ds(wid, 1), :, :], s_sems[slot])

    def wait_store(slot):
        for _half in range(PL):
            pltpu.make_async_copy(btile_v.at[pl.ds(0, 8), :, :, :],
                                  out_hbm.at[0, :, pl.ds(0, 1), :, :],
                                  s_sems[slot]).wait()

    lane = lax.broadcasted_iota(jnp.int32, (16,), 0)
    zeros16 = lane * 0

    def compute(l, slot, half):
        rbase = (slot * PL + half) * BBLK
        brow = (slot * PL + half) * 8
        pos_regs = [pos_v[l, pl.ds(k * 16, 16)] for k in range(HID // 16)]

        def add_body(j, _):
            for k in range(HID // 16):
                sl = pl.ds(k * 16, 16)
                rows_v[rbase + j, sl] = rows_v[rbase + j, sl] + pos_regs[k]
            return 0
        lax.fori_loop(0, BBLK, add_body, 0, unroll=4)

        def blk_body(t, _):
            jj = lax.shift_right_logical(t, 4)
            d = lax.bitwise_and(t, 15)
            jvec = jj * 16 + lane
            rowvec = rbase + jvec
            perm = lax.bitwise_and(lane + d, 15)
            perm_hi = lax.shift_right_logical(perm, 3)
            perm_lo = lax.bitwise_and(perm, 7)
            for k in range(HID // 16):
                v = plsc.load_gather(rows_v, [rowvec, k * 16 + perm])
                plsc.store_scatter(
                    btile_v,
                    [brow + k * 2 + perm_hi, zeros16, perm_lo, jvec],
                    v)
            return 0
        lax.fori_loop(0, (BBLK // 16) * 16, blk_body, 0, unroll=4)

    issue_gather(0, 0)

    NSTEP = L // PL

    def macro_body(i, _):
        for p in range(2):
            step = 2 * i + p
            slot = p
            other = 1 - p
            wait_gather(slot)
            @pl.when(i > 0)
            def _():
                wait_store(other)
            @pl.when(step + 1 < NSTEP)
            def _():
                issue_gather(step + 1, other)
            for half in range(PL):
                l = step * PL + half
                compute(l, slot, half)
                issue_store(l, slot, half)
        return 0

    lax.fori_loop(0, NSTEP // 2, macro_body, 0)

    for slot in range(NSLOT):
        wait_store(slot)


@jax.jit
def _tpe(xt, tok_table, pos_table):
    mesh = plsc.VectorSubcoreMesh(core_axis_name="c", subcore_axis_name="s")
    kern = functools.partial(
        pl.kernel,
        mesh=mesh,
        out_type=jax.ShapeDtypeStruct((L, 8, NUM_WORKERS, 8, 128),
                                      jnp.float32),
        scratch_types=[
            pltpu.VMEM((L // 2, BBLK), jnp.int32),
            pltpu.VMEM((L // PL, PL * BBLK), jnp.int32),
            pltpu.VMEM((L, HID), jnp.float32),
            pltpu.VMEM((NSLOT * PL * BBLK, HID), jnp.float32),
            pltpu.VMEM((NSLOT * PL * 8, 1, 8, 128), jnp.float32),
            pltpu.SemaphoreType.DMA,
            pltpu.SemaphoreType.DMA,
            pltpu.SemaphoreType.DMA,
            pltpu.SemaphoreType.DMA,
        ],
        compiler_params=pltpu.CompilerParams(use_tc_tiling_on_sc=False,
                                             needs_layout_passes=False),
    )(_sc_body)
    return kern(xt, tok_table, pos_table)


def kernel(x, tok_table, pos_table):
    xt = x.T.astype(jnp.int32)
    out5 = _tpe(xt, tok_table, pos_table)
    return out5.transpose(2, 4, 0, 1, 3).reshape(B, L, HID)
